# baseline (device time: 15631 ns/iter reference)
import jax
import jax.numpy as jnp
from jax import lax
from jax.experimental import pallas as pl
from jax.experimental.pallas import tpu as pltpu

N_DEV = 4
PIECES = 2


def kernel(A, B):
    m, k = A.shape
    _, n = B.shape
    m_out = m // N_DEV
    m_pc = m_out // PIECES

    def body(a_ref, b_ref, out_ref, send_ref, recv_ref, scl_send_ref,
             scl_recv_ref, b16_ref, send_sems, recv_sems,
             scl_send_sems, scl_recv_sems):
        my = lax.axis_index("i")

        barrier_sem = pltpu.get_barrier_semaphore()
        for off in (1, 2, 3):
            pl.semaphore_signal(
                barrier_sem, inc=1,
                device_id=((my + off) % N_DEV,),
                device_id_type=pl.DeviceIdType.MESH,
            )

        b16_ref[:, :] = b_ref[:, :].astype(jnp.bfloat16)

        rdmas = []
        first = True
        for half in range(PIECES):
            for off in (2, 1, 3):
                tgt = (my + off) % N_DEV
                slot = off - 1
                p = slot * PIECES + half
                rp = (3 - off) * PIECES + half
                part = jnp.dot(
                    a_ref[pl.ds(tgt * m_out + half * m_pc, m_pc), :]
                    .astype(jnp.bfloat16),
                    b16_ref[:, :],
                    preferred_element_type=jnp.float32,
                )
                absmax = jnp.maximum(jnp.max(jnp.abs(part)), 1e-30)
                send_ref[p] = jnp.clip(
                    jnp.round(part * (127.0 / absmax)), -127.0, 127.0
                ).astype(jnp.int8)
                scl_send_ref[p] = jnp.full(
                    (8, 128), absmax * (1.0 / 127.0), jnp.float32
                )
                if first:
                    pl.semaphore_wait(barrier_sem, 3)
                    first = False
                scl_rdma = pltpu.make_async_remote_copy(
                    src_ref=scl_send_ref.at[p],
                    dst_ref=scl_recv_ref.at[rp],
                    send_sem=scl_send_sems.at[p],
                    recv_sem=scl_recv_sems.at[rp],
                    device_id=(tgt,),
                    device_id_type=pl.DeviceIdType.MESH,
                )
                scl_rdma.start()
                rdma = pltpu.make_async_remote_copy(
                    src_ref=send_ref.at[p],
                    dst_ref=recv_ref.at[rp],
                    send_sem=send_sems.at[p],
                    recv_sem=recv_sems.at[rp],
                    device_id=(tgt,),
                    device_id_type=pl.DeviceIdType.MESH,
                )
                rdma.start()
                rdmas.append((rdma, scl_rdma, rp, half))

        out_ref[:, :] = jnp.dot(
            a_ref[pl.ds(my * m_out, m_out), :].astype(jnp.bfloat16),
            b16_ref[:, :],
            preferred_element_type=jnp.float32,
        )

        for half in range(PIECES):
            dqs = []
            for rdma, scl_rdma, rp, _ in rdmas[half * 3:(half + 1) * 3]:
                scl_rdma.wait_recv()
                rdma.wait_recv()
                dqs.append(
                    recv_ref[rp].astype(jnp.float32) * scl_recv_ref[rp][0, 0]
                )
            rows = pl.ds(half * m_pc, m_pc)
            out_ref[rows, :] = out_ref[rows, :] + (dqs[0] + dqs[1] + dqs[2])

        for rdma, scl_rdma, _, _ in rdmas:
            rdma.wait_send()
            scl_rdma.wait_send()


    n_pc = (N_DEV - 1) * PIECES
    return pl.pallas_call(
        body,
        out_shape=jax.ShapeDtypeStruct((m_out, n), jnp.float32),
        in_specs=[
            pl.BlockSpec(memory_space=pltpu.VMEM),
            pl.BlockSpec(memory_space=pltpu.VMEM),
        ],
        out_specs=pl.BlockSpec(memory_space=pltpu.VMEM),
        scratch_shapes=[
            pltpu.VMEM((n_pc, m_pc, n), jnp.int8),
            pltpu.VMEM((n_pc, m_pc, n), jnp.int8),
            pltpu.VMEM((n_pc, 8, 128), jnp.float32),
            pltpu.VMEM((n_pc, 8, 128), jnp.float32),
            pltpu.VMEM((k, n), jnp.bfloat16),
            pltpu.SemaphoreType.DMA((n_pc,)),
            pltpu.SemaphoreType.DMA((n_pc,)),
            pltpu.SemaphoreType.DMA((n_pc,)),
            pltpu.SemaphoreType.DMA((n_pc,)),
        ],
        compiler_params=pltpu.CompilerParams(collective_id=0),
    )(A, B)
